# baseline (speedup 1.0000x reference)
import functools

import numpy as np
import jax
import jax.numpy as jnp
from jax import lax
from jax.experimental import pallas as pl
from jax.experimental.pallas import tpu as pltpu
from jax.experimental.pallas import tpu_sc as plsc

VOCAB = 100000
EMBED = 128
MAXLEN = 512
B = 1024
L = 200

NUM_WORKERS = 32
ROWS_PER_W = B * L // NUM_WORKERS
CHUNK = L
N_CHUNKS = ROWS_PER_W // CHUNK
LANES = 16
GROUPS = EMBED // LANES
NBUF = 4
OUTER = N_CHUNKS // NBUF


def _make_pe():
    pe = np.zeros((MAXLEN, EMBED), dtype=np.float32)
    position = np.arange(0, MAXLEN)[:, np.newaxis]
    div_term = np.exp(np.arange(0, EMBED, 2) * -(np.log(10000.0) / EMBED))
    pe[:, 0::2] = np.sin(position * div_term)
    pe[:, 1::2] = np.cos(position * div_term)
    return jnp.asarray(pe[:L, :])


_MESH = plsc.VectorSubcoreMesh(core_axis_name="c", subcore_axis_name="s")


@functools.partial(
    pl.kernel,
    mesh=_MESH,
    out_type=jax.ShapeDtypeStruct((B * L, EMBED), jnp.float32),
    scratch_types=(
        [pltpu.VMEM((CHUNK,), jnp.int32) for _ in range(NBUF)]
        + [pltpu.VMEM((CHUNK, EMBED), jnp.float32) for _ in range(NBUF)]
        + [pltpu.VMEM((L, EMBED), jnp.float32)]
        + [pltpu.SemaphoreType.DMA for _ in range(3 * NBUF)]
    ),
)
def _emb_lookup(x_hbm, w_hbm, pe_hbm, out_hbm, *scratch):
    idx_v = scratch[0:NBUF]
    rows_v = scratch[NBUF:2 * NBUF]
    pe_v = scratch[2 * NBUF]
    sem_idx = scratch[2 * NBUF + 1:3 * NBUF + 1]
    sem_in = scratch[3 * NBUF + 1:4 * NBUF + 1]
    sem_out = scratch[4 * NBUF + 1:5 * NBUF + 1]

    wid = lax.axis_index("s") * 2 + lax.axis_index("c")
    base = wid * ROWS_PER_W

    def idx_copy(c, b):
        return pltpu.make_async_copy(
            x_hbm.at[pl.ds(base + c * CHUNK, CHUNK)], idx_v[b], sem_idx[b])

    def gather(b):
        return pltpu.make_async_copy(w_hbm.at[idx_v[b]], rows_v[b], sem_in[b])

    def scatter(c, b):
        return pltpu.make_async_copy(
            rows_v[b], out_hbm.at[pl.ds(base + c * CHUNK, CHUNK)], sem_out[b])

    pltpu.sync_copy(pe_hbm, pe_v)

    idx_copy(0, 0).start()
    idx_copy(1, 1).start()
    idx_copy(2, 2).start()
    idx_copy(0, 0).wait()
    gather(0).start()
    idx_copy(1, 1).wait()
    gather(1).start()

    def outer_body(i, carry):
        for b in range(NBUF):
            c = i * NBUF + b
            b2 = (b + 2) % NBUF
            b3 = (b + 3) % NBUF

            if b >= 2:
                scatter(c - 2, b2).wait()
            else:
                @pl.when(i >= 1)
                def _():
                    scatter(c - 2, b2).wait()

            def start_gather():
                idx_copy(c + 2, b2).wait()
                gather(b2).start()
            if b < 2:
                start_gather()
            else:
                pl.when(i < OUTER - 1)(start_gather)

            def start_idx():
                idx_copy(c + 3, b3).start()
            if b == 0:
                start_idx()
            else:
                pl.when(i < OUTER - 1)(start_idx)

            gather(b).wait()
            rv = rows_v[b]

            def row_body(r, rcarry):
                for g in range(GROUPS):
                    sl = pl.ds(g * LANES, LANES)
                    rv[r, sl] = rv[r, sl] + pe_v[r, sl]
                return rcarry

            lax.fori_loop(0, CHUNK, row_body, 0, unroll=2)
            scatter(c, b).start()
        return carry

    lax.fori_loop(0, OUTER, outer_body, 0)

    scatter(N_CHUNKS - 2, (N_CHUNKS - 2) % NBUF).wait()
    scatter(N_CHUNKS - 1, (N_CHUNKS - 1) % NBUF).wait()


def kernel(x, W):
    pe = _make_pe()
    out = _emb_lookup(x.reshape(-1), W, pe)
    return out.reshape(B, L, EMBED)


# speedup vs baseline: 2.5768x; 2.5768x over previous
import functools

import numpy as np
import jax
import jax.numpy as jnp
from jax import lax
from jax.experimental import pallas as pl
from jax.experimental.pallas import tpu as pltpu
from jax.experimental.pallas import tpu_sc as plsc

VOCAB = 100000
EMBED = 128
MAXLEN = 512
B = 1024
L = 200

NUM_WORKERS = 32
ROWS_PER_W = B * L // NUM_WORKERS
CHUNK = L
N_CHUNKS = ROWS_PER_W // CHUNK
LANES = 16
GROUPS = EMBED // LANES
NBUF = 3
OUTER = (N_CHUNKS - 2) // NBUF


def _make_pe():
    pe = np.zeros((MAXLEN, EMBED), dtype=np.float32)
    position = np.arange(0, MAXLEN)[:, np.newaxis]
    div_term = np.exp(np.arange(0, EMBED, 2) * -(np.log(10000.0) / EMBED))
    pe[:, 0::2] = np.sin(position * div_term)
    pe[:, 1::2] = np.cos(position * div_term)
    return jnp.asarray(pe[:L, :])


_MESH = plsc.VectorSubcoreMesh(core_axis_name="c", subcore_axis_name="s")


@functools.partial(
    pl.kernel,
    mesh=_MESH,
    out_type=jax.ShapeDtypeStruct((B * L, EMBED), jnp.float32),
    scratch_types=(
        [pltpu.VMEM((ROWS_PER_W,), jnp.int32)]
        + [pltpu.VMEM((CHUNK, EMBED), jnp.float32) for _ in range(NBUF)]
        + [pltpu.VMEM((L, EMBED), jnp.float32)]
        + [pltpu.SemaphoreType.DMA for _ in range(2 * NBUF)]
    ),
)
def _emb_lookup(x_hbm, w_hbm, pe_hbm, out_hbm, idx_v, r0, r1, r2, pe_v, *sems):
    rows_v = (r0, r1, r2)
    sem_in = sems[0:NBUF]
    sem_out = sems[NBUF:2 * NBUF]

    wid = lax.axis_index("s") * 2 + lax.axis_index("c")
    base = wid * ROWS_PER_W

    def gather(c, b):
        return pltpu.make_async_copy(
            w_hbm.at[idx_v.at[pl.ds(c * CHUNK, CHUNK)]], rows_v[b], sem_in[b])

    def writeback(c, b):
        return pltpu.make_async_copy(
            rows_v[b], out_hbm.at[pl.ds(base + c * CHUNK, CHUNK)], sem_out[b])

    pltpu.sync_copy(x_hbm.at[pl.ds(base, ROWS_PER_W)], idx_v)
    pltpu.sync_copy(pe_hbm, pe_v)

    def add_pe(b):
        rv = rows_v[b]

        def row_body(r, rcarry):
            for g in range(GROUPS):
                sl = pl.ds(g * LANES, LANES)
                rv[r, sl] = rv[r, sl] + pe_v[r, sl]
            return rcarry

        lax.fori_loop(0, CHUNK, row_body, 0)

    gather(0, 0).start()
    gather(1, 1).start()
    gather(0, 0).wait()
    add_pe(0)
    writeback(0, 0).start()
    gather(2, 2).start()
    gather(1, 1).wait()
    add_pe(1)
    writeback(1, 1).start()

    def outer_body(i, carry):
        for k in range(NBUF):
            c = NBUF * i + 2 + k
            b = (2 + k) % NBUF
            bn = (3 + k) % NBUF
            writeback(c - 2, bn).wait()
            if k == NBUF - 1:
                @pl.when(i < OUTER - 1)
                def _():
                    gather(c + 1, bn).start()
            else:
                gather(c + 1, bn).start()
            gather(c, b).wait()
            add_pe(b)
            writeback(c, b).start()
        return carry

    lax.fori_loop(0, OUTER, outer_body, 0)

    writeback(N_CHUNKS - 2, (N_CHUNKS - 2) % NBUF).wait()
    writeback(N_CHUNKS - 1, (N_CHUNKS - 1) % NBUF).wait()


def kernel(x, W):
    pe = _make_pe()
    out = _emb_lookup(x.reshape(-1), W, pe)
    return out.reshape(B, L, EMBED)
